# baseline (speedup 1.0000x reference)
import functools

import jax
import jax.numpy as jnp
from jax import lax
from jax.experimental import pallas as pl
from jax.experimental.pallas import tpu as pltpu

_DROPOUT_P = 0.5
_DROPOUT_SEED = 1234
_GOLDEN = 0x9E3779B9


def _relu_dropout(acc, bias, row0, col0, n_total, seed_u, scale):
    tm, tn = acc.shape
    y = jnp.maximum(acc + bias, 0.0)
    base = jnp.uint32(row0 * n_total) + jnp.uint32(col0)
    lin = (lax.broadcasted_iota(jnp.int32, (tm, tn), 0) * n_total
           + lax.broadcasted_iota(jnp.int32, (tm, tn), 1)).astype(jnp.uint32)
    h = (lin + base) ^ jnp.uint32(seed_u)
    h = h ^ (h >> 16)
    h = h * jnp.uint32(0x85EBCA6B)
    h = h ^ (h >> 13)
    h = h * jnp.uint32(0xC2B2AE35)
    keep = (h & jnp.uint32(0x00800000)) != 0
    return jnp.where(keep, y * jnp.float32(scale), 0.0)


def _fused_kernel(x_ref, w_hbm, b_ref, o_ref, wb_ref, wstg_ref, sem,
                  *, n_total, seed_u, scale, ch, nc):
    i = pl.program_id(0)
    tm = o_ref.shape[0]
    row0 = i * tm
    xq = x_ref[...].astype(jnp.bfloat16)

    def start_copy(c):
        pltpu.make_async_copy(
            w_hbm.at[pl.ds(c * ch, ch), :], wstg_ref.at[c % 2],
            sem.at[c % 2]).start()

    def wait_copy(c):
        pltpu.make_async_copy(
            w_hbm.at[pl.ds(c * ch, ch), :], wstg_ref.at[c % 2],
            sem.at[c % 2]).wait()

    @pl.when(i == 0)
    def _():
        start_copy(0)
        if nc > 1:
            start_copy(1)
        for c in range(nc):
            wait_copy(c)
            wc = wstg_ref[c % 2].astype(jnp.bfloat16)
            wb_ref[pl.ds(c * ch, ch), :] = wc
            if c + 2 < nc:
                start_copy(c + 2)
            acc = lax.dot_general(
                xq, wc, dimension_numbers=(((1,), (1,)), ((), ())),
                preferred_element_type=jnp.float32)
            o_ref[:, pl.ds(c * ch, ch)] = _relu_dropout(
                acc, b_ref[:, pl.ds(c * ch, ch)], row0, c * ch, n_total,
                seed_u, scale)

    @pl.when(i > 0)
    def _():
        acc = lax.dot_general(
            xq, wb_ref[...], dimension_numbers=(((1,), (1,)), ((), ())),
            preferred_element_type=jnp.float32)
        o_ref[...] = _relu_dropout(acc, b_ref[...], row0, 0, n_total,
                                   seed_u, scale)


def kernel(x, w, b):
    B, K = x.shape
    N, Kw = w.shape
    assert Kw == K

    bm = min(256, B)
    ch = min(256, N)
    nc = N // ch
    grid = (B // bm,)

    b2 = b.reshape(1, N).astype(jnp.float32)

    seed_u = (_DROPOUT_SEED * _GOLDEN) & 0xFFFFFFFF
    body = functools.partial(
        _fused_kernel, n_total=N, seed_u=seed_u,
        scale=1.0 / (1.0 - _DROPOUT_P), ch=ch, nc=nc)

    out = pl.pallas_call(
        body,
        grid=grid,
        in_specs=[
            pl.BlockSpec((bm, K), lambda i: (i, 0)),
            pl.BlockSpec(memory_space=pl.ANY),
            pl.BlockSpec((1, N), lambda i: (0, 0)),
        ],
        out_specs=pl.BlockSpec((bm, N), lambda i: (i, 0)),
        out_shape=jax.ShapeDtypeStruct((B, N), jnp.float32),
        scratch_shapes=[
            pltpu.VMEM((N, K), jnp.bfloat16),
            pltpu.VMEM((2, ch, K), jnp.float32),
            pltpu.SemaphoreType.DMA((2,)),
        ],
        compiler_params=pltpu.CompilerParams(
            dimension_semantics=("arbitrary",),
            vmem_limit_bytes=64 * 1024 * 1024),
    )(x, w, b2)
    return out


# speedup vs baseline: 1.0154x; 1.0154x over previous
import functools

import jax
import jax.numpy as jnp
from jax import lax
from jax.experimental import pallas as pl
from jax.experimental.pallas import tpu as pltpu

_DROPOUT_P = 0.5
_DROPOUT_SEED = 1234
_GOLDEN = 0x9E3779B9


def _fused_kernel(x_hbm, w_ref, b_ref, o_ref, xb_ref, xstg_ref, lin_ref, sem,
                  *, n_total, seed_u, scale, ch, nc):
    i = pl.program_id(0)
    j = pl.program_id(1)
    tm, tn = o_ref.shape

    def start_copy(c):
        pltpu.make_async_copy(
            x_hbm.at[pl.ds(i * tm + c * ch, ch), :], xstg_ref.at[c % 2],
            sem.at[c % 2]).start()

    def wait_copy(c):
        pltpu.make_async_copy(
            x_hbm.at[pl.ds(i * tm + c * ch, ch), :], xstg_ref.at[c % 2],
            sem.at[c % 2]).wait()

    first = jnp.logical_and(i == 0, j == 0)

    @pl.when(first)
    def _():
        lin_ref[...] = (lax.broadcasted_iota(jnp.int32, (tm, tn), 0) * n_total
                        + lax.broadcasted_iota(jnp.int32, (tm, tn), 1)
                        ).astype(jnp.uint32)

    @pl.when(j == 0)
    def _():
        start_copy(0)
        if nc > 1:
            start_copy(1)
        for c in range(nc):
            wait_copy(c)
            xb_ref[pl.ds(c * ch, ch), :] = xstg_ref[c % 2].astype(jnp.bfloat16)
            if c + 2 < nc:
                start_copy(c + 2)

    acc = lax.dot_general(
        xb_ref[...], w_ref[...].astype(jnp.bfloat16),
        dimension_numbers=(((1,), (1,)), ((), ())),
        preferred_element_type=jnp.float32)
    y = jnp.maximum(acc + b_ref[...], 0.0)

    base = ((i * tm) * n_total + j * tn).astype(jnp.uint32)
    h = (lin_ref[...] + base) ^ jnp.uint32(seed_u)
    h = h ^ (h >> 16)
    h = h * jnp.uint32(0x85EBCA6B)
    h = h ^ (h >> 13)
    h = h * jnp.uint32(0xC2B2AE35)
    keep = (h & jnp.uint32(0x00800000)) != 0
    o_ref[...] = jnp.where(keep, y * jnp.float32(scale), 0.0)


def kernel(x, w, b):
    B, K = x.shape
    N, Kw = w.shape
    assert Kw == K

    bm = min(2048, B)
    bn = min(512, N)
    ch = min(256, bm)
    nc = bm // ch
    grid = (B // bm, N // bn)

    b2 = b.reshape(1, N).astype(jnp.float32)

    seed_u = (_DROPOUT_SEED * _GOLDEN) & 0xFFFFFFFF
    body = functools.partial(
        _fused_kernel, n_total=N, seed_u=seed_u,
        scale=1.0 / (1.0 - _DROPOUT_P), ch=ch, nc=nc)

    out = pl.pallas_call(
        body,
        grid=grid,
        in_specs=[
            pl.BlockSpec(memory_space=pl.ANY),
            pl.BlockSpec((bn, K), lambda i, j: (j, 0)),
            pl.BlockSpec((1, bn), lambda i, j: (0, j)),
        ],
        out_specs=pl.BlockSpec((bm, bn), lambda i, j: (i, j)),
        out_shape=jax.ShapeDtypeStruct((B, N), jnp.float32),
        scratch_shapes=[
            pltpu.VMEM((bm, K), jnp.bfloat16),
            pltpu.VMEM((2, ch, K), jnp.float32),
            pltpu.VMEM((bm, bn), jnp.uint32),
            pltpu.SemaphoreType.DMA((2,)),
        ],
        compiler_params=pltpu.CompilerParams(
            dimension_semantics=("arbitrary", "arbitrary"),
            vmem_limit_bytes=64 * 1024 * 1024),
    )(x, w, b2)
    return out


# speedup vs baseline: 1.0442x; 1.0284x over previous
import functools

import jax
import jax.numpy as jnp
from jax import lax
from jax.experimental import pallas as pl
from jax.experimental.pallas import tpu as pltpu

_DROPOUT_P = 0.5
_DROPOUT_SEED = 1234
_GOLDEN = 0x9E3779B9


def _fused_kernel(x_ref, w_ref, b_ref, o_ref, xb_ref, lin_ref, *, n_total,
                  seed_u, scale):
    j = pl.program_id(1)
    tm, tn = o_ref.shape
    kdim = x_ref.shape[1]
    kh = kdim // 2
    first = jnp.logical_and(pl.program_id(0) == 0, j == 0)

    @pl.when(first)
    def _():
        lin_ref[...] = (lax.broadcasted_iota(jnp.int32, (tm, tn), 0) * n_total
                        + lax.broadcasted_iota(jnp.int32, (tm, tn), 1)
                        ).astype(jnp.uint32)

    @pl.when(j == 0)
    def _():
        xb_ref[:, :kh] = x_ref[:, :kh].astype(jnp.bfloat16)
        xb_ref[:, kh:] = x_ref[:, kh:].astype(jnp.bfloat16)

    base = ((pl.program_id(0) * tm) * n_total + j * tn).astype(jnp.uint32)
    xb = xb_ref[...]
    half = tn // 2
    for h0 in range(2):
        sl = pl.ds(h0 * half, half)
        wc = w_ref[sl, :].astype(jnp.bfloat16)
        acc = (lax.dot_general(
            xb[:, :kh], wc[:, :kh],
            dimension_numbers=(((1,), (1,)), ((), ())),
            preferred_element_type=jnp.float32)
            + lax.dot_general(
            xb[:, kh:], wc[:, kh:],
            dimension_numbers=(((1,), (1,)), ((), ())),
            preferred_element_type=jnp.float32))
        y = jnp.maximum(acc + b_ref[:, sl], 0.0)
        h = (lin_ref[:, sl] + base) ^ jnp.uint32(seed_u)
        h = h ^ (h >> 16)
        h = h * jnp.uint32(0x85EBCA6B)
        h = h ^ (h >> 13)
        h = h * jnp.uint32(0xC2B2AE35)
        keep = (h & jnp.uint32(0x00800000)) != 0
        o_ref[:, sl] = jnp.where(keep, y * jnp.float32(scale), 0.0)


def kernel(x, w, b):
    B, K = x.shape
    N, Kw = w.shape
    assert Kw == K

    bm = min(1024, B)
    bn = min(512, N)
    grid = (B // bm, N // bn)

    b2 = b.reshape(1, N).astype(jnp.float32)

    seed_u = (_DROPOUT_SEED * _GOLDEN) & 0xFFFFFFFF
    body = functools.partial(
        _fused_kernel, n_total=N, seed_u=seed_u,
        scale=1.0 / (1.0 - _DROPOUT_P))

    out = pl.pallas_call(
        body,
        grid=grid,
        in_specs=[
            pl.BlockSpec((bm, K), lambda i, j: (i, 0)),
            pl.BlockSpec((bn, K), lambda i, j: (j, 0)),
            pl.BlockSpec((1, bn), lambda i, j: (0, j)),
        ],
        out_specs=pl.BlockSpec((bm, bn), lambda i, j: (i, j)),
        out_shape=jax.ShapeDtypeStruct((B, N), jnp.float32),
        scratch_shapes=[
            pltpu.VMEM((bm, K), jnp.bfloat16),
            pltpu.VMEM((bm, bn), jnp.uint32),
        ],
        compiler_params=pltpu.CompilerParams(
            dimension_semantics=("arbitrary", "arbitrary"),
            vmem_limit_bytes=64 * 1024 * 1024),
    )(x, w, b2)
    return out
